# baseline (speedup 1.0000x reference)
import jax
import jax.numpy as jnp
import numpy as np
from jax import lax
from jax.experimental import pallas as pl
from jax.experimental.pallas import tpu as pltpu
from jax.experimental.pallas import tpu_sc as plsc

BATCH = 4
MODEL_DIM = 1024
MAX_LEN = 2048

NC = 2
NS = 16
LANES = 16
NW = NC * NS

B_TOTAL = BATCH * MAX_LEN
T_PER_W = MAX_LEN // NW
CHUNK = 16
QR = 8
QPC = CHUNK // QR
N_TC = T_PER_W // CHUNK
N_CHUNKS = N_TC * BATCH
NB = 5
GLEAD = 3
NQ = MAX_LEN // QR

_VR = MODEL_DIM // LANES

_OFF_A1 = 0
_OFF_A2 = NQ * MODEL_DIM
_OFF_B1 = 2 * NQ * MODEL_DIM
_OFF_B2 = 2 * NQ * MODEL_DIM + QR * MODEL_DIM


def _aux_tables_np():
    i = np.arange(MODEL_DIM, dtype=np.float64)
    w = 1.0 / (10000.0 ** (2.0 * i / MODEL_DIM))
    even = (np.arange(MODEL_DIM) % 2) == 0
    th = QR * np.arange(NQ, dtype=np.float64).reshape(-1, 1) * w
    a1 = np.where(even, np.sin(th), np.cos(th)).astype(np.float32)
    a2 = np.where(even, np.cos(th), -np.sin(th)).astype(np.float32)
    rw = np.arange(QR, dtype=np.float64).reshape(-1, 1) * w
    b1 = np.cos(rw).astype(np.float32)
    b2 = np.sin(rw).astype(np.float32)
    return np.concatenate(
        [a1.reshape(-1), a2.reshape(-1), b1.reshape(-1), b2.reshape(-1)]
    )


_AUX = _aux_tables_np()


def _sc_body(
    table_hbm, idx_hbm, aux_hbm, out_hbm,
    idx_v, rows_all, posrec, b_v, a_all, gsem, psem, asem, bsem, isem,
):
    rows = [rows_all.at[j] for j in range(NB)]
    a_v = [a_all.at[k] for k in range(2)]

    wid = lax.axis_index("s") * NC + lax.axis_index("c")
    t0 = wid * T_PER_W

    bd = pltpu.async_copy(
        aux_hbm.at[pl.ds(_OFF_B1, 2 * QR * MODEL_DIM)], b_v, bsem
    )

    def issue_a(tc):
        q0 = (wid * N_TC + tc) * QPC
        buf = tc % 2
        d1 = pltpu.async_copy(
            aux_hbm.at[pl.ds(_OFF_A1 + q0 * MODEL_DIM, QPC * MODEL_DIM)],
            a_v[buf].at[pl.ds(0, QPC * MODEL_DIM)],
            asem.at[buf],
        )
        d2 = pltpu.async_copy(
            aux_hbm.at[pl.ds(_OFF_A2 + q0 * MODEL_DIM, QPC * MODEL_DIM)],
            a_v[buf].at[pl.ds(QPC * MODEL_DIM, QPC * MODEL_DIM)],
            asem.at[buf],
        )
        return (d1, d2)

    a_desc = [None] * N_TC
    a_desc[0] = issue_a(0)
    a_desc[1] = issue_a(1)

    idx_descs = [
        pltpu.async_copy(
            idx_hbm.at[b, pl.ds(t0, T_PER_W)],
            idx_v.at[pl.ds(b * T_PER_W, T_PER_W)],
            isem,
        )
        for b in range(BATCH)
    ]
    for d in idx_descs:
        d.wait()

    def issue_gather(n):
        tc, b = n // BATCH, n % BATCH
        return pltpu.async_copy(
            table_hbm.at[idx_v.at[pl.ds(b * T_PER_W + tc * CHUNK, CHUNK)]],
            rows[n % NB],
            gsem.at[n % NB],
        )

    gat = [None] * NB
    put = [None] * NB
    for n in range(GLEAD):
        gat[n % NB] = issue_gather(n)

    for c in range(N_CHUNKS):
        j = c % NB
        tc, b = c // BATCH, c % BATCH

        if c == BATCH:
            a_desc[2] = issue_a(2)
        if c == 2 * BATCH:
            a_desc[3] = issue_a(3)

        n = c + GLEAD
        if n < N_CHUNKS:
            if n >= NB:
                put[n % NB].wait()
            gat[n % NB] = issue_gather(n)

        if b == 0:
            if c == 0:
                bd.wait()
            a_desc[tc][0].wait()
            a_desc[tc][1].wait()
            a_tc = a_v[tc % 2]

            @plsc.parallel_loop(0, _VR, unroll=2)
            def gen_body(jc):
                off = pl.multiple_of(jc << 4, LANES)
                a1q0 = a_tc[pl.ds(off, LANES)]
                a1q1 = a_tc[pl.ds(MODEL_DIM + off, LANES)]
                a2q0 = a_tc[pl.ds(2 * MODEL_DIM + off, LANES)]
                a2q1 = a_tc[pl.ds(3 * MODEL_DIM + off, LANES)]
                for r in range(QR):
                    b1r = b_v[pl.ds(r * MODEL_DIM + off, LANES)]
                    b2r = b_v[pl.ds((QR + r) * MODEL_DIM + off, LANES)]
                    posrec[r, pl.ds(off, LANES)] = a1q0 * b1r + a2q0 * b2r
                    posrec[QR + r, pl.ds(off, LANES)] = a1q1 * b1r + a2q1 * b2r

        gat[j].wait()
        rows_j = rows[j]

        @plsc.parallel_loop(0, CHUNK * _VR, unroll=8)
        def add_body(i):
            r = i >> 6
            off = pl.multiple_of((i & (_VR - 1)) << 4, LANES)
            rows_j[r, pl.ds(off, LANES)] = (
                rows_j[r, pl.ds(off, LANES)] + posrec[r, pl.ds(off, LANES)]
            )

        put[j] = pltpu.async_copy(
            rows_j,
            out_hbm.at[pl.ds(b * MAX_LEN + t0 + tc * CHUNK, CHUNK)],
            psem.at[j],
        )

    for c in range(N_CHUNKS - NB, N_CHUNKS):
        put[c % NB].wait()


@jax.jit
def _embed(idx, table, aux):
    mesh = plsc.VectorSubcoreMesh(
        core_axis_name="c", subcore_axis_name="s", num_cores=NC, num_subcores=NS
    )
    scratch = [
        pltpu.VMEM((BATCH * T_PER_W,), jnp.int32),
        pltpu.VMEM((NB, CHUNK, MODEL_DIM), jnp.float32),
        pltpu.VMEM((CHUNK, MODEL_DIM), jnp.float32),
        pltpu.VMEM((2 * QR * MODEL_DIM,), jnp.float32),
        pltpu.VMEM((2, 2 * QPC * MODEL_DIM), jnp.float32),
        pltpu.SemaphoreType.DMA((NB,)),
        pltpu.SemaphoreType.DMA((NB,)),
        pltpu.SemaphoreType.DMA((2,)),
        pltpu.SemaphoreType.DMA,
        pltpu.SemaphoreType.DMA,
    ]
    fn = pl.kernel(
        _sc_body,
        out_type=jax.ShapeDtypeStruct((B_TOTAL, MODEL_DIM), jnp.float32),
        mesh=mesh,
        scratch_types=scratch,
    )
    return fn(table, idx, aux)


def kernel(x, table):
    idx = x.astype(jnp.int32)
    out = _embed(idx, table, jnp.asarray(_AUX))
    return out.reshape(BATCH, MAX_LEN, MODEL_DIM)


# speedup vs baseline: 1.0528x; 1.0528x over previous
import jax
import jax.numpy as jnp
import numpy as np
from jax import lax
from jax.experimental import pallas as pl
from jax.experimental.pallas import tpu as pltpu
from jax.experimental.pallas import tpu_sc as plsc

BATCH = 4
MODEL_DIM = 1024
MAX_LEN = 2048

NC = 2
NS = 16
LANES = 16
NW = NC * NS

B_TOTAL = BATCH * MAX_LEN
T_PER_W = MAX_LEN // NW
CHUNK = 16
QR = 8
QPC = CHUNK // QR
N_TC = T_PER_W // CHUNK
N_CHUNKS = N_TC * BATCH
NB = 5
GLEAD = 3
NQ = MAX_LEN // QR

_VR = MODEL_DIM // LANES

_OFF_A1 = 0
_OFF_A2 = NQ * MODEL_DIM
_OFF_B1 = 2 * NQ * MODEL_DIM
_OFF_B2 = 2 * NQ * MODEL_DIM + QR * MODEL_DIM


def _aux_tables_np():
    i = np.arange(MODEL_DIM, dtype=np.float64)
    w = 1.0 / (10000.0 ** (2.0 * i / MODEL_DIM))
    even = (np.arange(MODEL_DIM) % 2) == 0
    th = QR * np.arange(NQ, dtype=np.float64).reshape(-1, 1) * w
    a1 = np.where(even, np.sin(th), np.cos(th)).astype(np.float32)
    a2 = np.where(even, np.cos(th), -np.sin(th)).astype(np.float32)
    rw = np.arange(QR, dtype=np.float64).reshape(-1, 1) * w
    b1 = np.cos(rw).astype(np.float32)
    b2 = np.sin(rw).astype(np.float32)
    return np.concatenate(
        [a1.reshape(-1), a2.reshape(-1), b1.reshape(-1), b2.reshape(-1)]
    )


_AUX = _aux_tables_np()


def _sc_body(
    table_hbm, idx_hbm, aux_hbm, out_hbm,
    idx_v, rows_all, posrec, b_v, a_all, gsem, psem, asem, bsem, isem,
):
    rows = [rows_all.at[j] for j in range(NB)]
    a_v = [a_all.at[k] for k in range(2)]

    wid = lax.axis_index("s") * NC + lax.axis_index("c")
    t0 = wid * T_PER_W

    bd = pltpu.async_copy(
        aux_hbm.at[pl.ds(_OFF_B1, 2 * QR * MODEL_DIM)], b_v, bsem
    )

    def issue_a(tc):
        q0 = (wid * N_TC + tc) * QPC
        buf = tc % 2
        d1 = pltpu.async_copy(
            aux_hbm.at[pl.ds(_OFF_A1 + q0 * MODEL_DIM, QPC * MODEL_DIM)],
            a_v[buf].at[pl.ds(0, QPC * MODEL_DIM)],
            asem.at[buf],
        )
        d2 = pltpu.async_copy(
            aux_hbm.at[pl.ds(_OFF_A2 + q0 * MODEL_DIM, QPC * MODEL_DIM)],
            a_v[buf].at[pl.ds(QPC * MODEL_DIM, QPC * MODEL_DIM)],
            asem.at[buf],
        )
        return (d1, d2)

    a_desc = [None] * N_TC
    a_desc[0] = issue_a(0)
    a_desc[1] = issue_a(1)

    idx_descs = [
        pltpu.async_copy(
            idx_hbm.at[b, pl.ds(t0, T_PER_W)],
            idx_v.at[pl.ds(b * T_PER_W, T_PER_W)],
            isem,
        )
        for b in range(BATCH)
    ]
    for d in idx_descs:
        d.wait()

    def issue_gather(n):
        tc, b = n // BATCH, n % BATCH
        return pltpu.async_copy(
            table_hbm.at[idx_v.at[pl.ds(b * T_PER_W + tc * CHUNK, CHUNK)]],
            rows[n % NB],
            gsem.at[n % NB],
        )

    gat = [None] * NB
    put = [None] * NB
    for n in range(GLEAD):
        gat[n % NB] = issue_gather(n)

    for c in range(N_CHUNKS):
        j = c % NB
        tc, b = c // BATCH, c % BATCH

        if c == BATCH:
            a_desc[2] = issue_a(2)
        if c == 2 * BATCH:
            a_desc[3] = issue_a(3)

        n = c + GLEAD
        if n < N_CHUNKS:
            if n >= NB:
                put[n % NB].wait()
            gat[n % NB] = issue_gather(n)

        if b == 0:
            if c == 0:
                bd.wait()
            a_desc[tc][0].wait()
            a_desc[tc][1].wait()
            a_tc = a_v[tc % 2]

            @plsc.parallel_loop(0, _VR, unroll=1)
            def gen_body(jc):
                off = pl.multiple_of(jc << 4, LANES)
                a1q0 = a_tc[pl.ds(off, LANES)]
                a1q1 = a_tc[pl.ds(MODEL_DIM + off, LANES)]
                a2q0 = a_tc[pl.ds(2 * MODEL_DIM + off, LANES)]
                a2q1 = a_tc[pl.ds(3 * MODEL_DIM + off, LANES)]
                for r in range(QR):
                    b1r = b_v[pl.ds(r * MODEL_DIM + off, LANES)]
                    b2r = b_v[pl.ds((QR + r) * MODEL_DIM + off, LANES)]
                    posrec[r, pl.ds(off, LANES)] = a1q0 * b1r + a2q0 * b2r
                    posrec[QR + r, pl.ds(off, LANES)] = a1q1 * b1r + a2q1 * b2r

        gat[j].wait()
        rows_j = rows[j]

        @plsc.parallel_loop(0, CHUNK * _VR, unroll=4)
        def add_body(i):
            r = i >> 6
            off = pl.multiple_of((i & (_VR - 1)) << 4, LANES)
            rows_j[r, pl.ds(off, LANES)] = (
                rows_j[r, pl.ds(off, LANES)] + posrec[r, pl.ds(off, LANES)]
            )

        put[j] = pltpu.async_copy(
            rows_j,
            out_hbm.at[pl.ds(b * MAX_LEN + t0 + tc * CHUNK, CHUNK)],
            psem.at[j],
        )

    for c in range(N_CHUNKS - NB, N_CHUNKS):
        put[c % NB].wait()


@jax.jit
def _embed(idx, table, aux):
    mesh = plsc.VectorSubcoreMesh(
        core_axis_name="c", subcore_axis_name="s", num_cores=NC, num_subcores=NS
    )
    scratch = [
        pltpu.VMEM((BATCH * T_PER_W,), jnp.int32),
        pltpu.VMEM((NB, CHUNK, MODEL_DIM), jnp.float32),
        pltpu.VMEM((CHUNK, MODEL_DIM), jnp.float32),
        pltpu.VMEM((2 * QR * MODEL_DIM,), jnp.float32),
        pltpu.VMEM((2, 2 * QPC * MODEL_DIM), jnp.float32),
        pltpu.SemaphoreType.DMA((NB,)),
        pltpu.SemaphoreType.DMA((NB,)),
        pltpu.SemaphoreType.DMA((2,)),
        pltpu.SemaphoreType.DMA,
        pltpu.SemaphoreType.DMA,
    ]
    fn = pl.kernel(
        _sc_body,
        out_type=jax.ShapeDtypeStruct((B_TOTAL, MODEL_DIM), jnp.float32),
        mesh=mesh,
        scratch_types=scratch,
    )
    return fn(table, idx, aux)


def kernel(x, table):
    idx = x.astype(jnp.int32)
    out = _embed(idx, table, jnp.asarray(_AUX))
    return out.reshape(BATCH, MAX_LEN, MODEL_DIM)


# speedup vs baseline: 1.1211x; 1.0649x over previous
import jax
import jax.numpy as jnp
import numpy as np
from jax import lax
from jax.experimental import pallas as pl
from jax.experimental.pallas import tpu as pltpu
from jax.experimental.pallas import tpu_sc as plsc

BATCH = 4
MODEL_DIM = 1024
MAX_LEN = 2048

NC = 2
NS = 16
LANES = 16
NW = NC * NS

B_TOTAL = BATCH * MAX_LEN
T_PER_W = MAX_LEN // NW
CHUNK = 16
QR = 8
QPC = CHUNK // QR
N_TC = T_PER_W // CHUNK
N_CHUNKS = N_TC * BATCH
NB = 5
GLEAD = 3
NQ = MAX_LEN // QR

_VR = MODEL_DIM // LANES

_OFF_A1 = 0
_OFF_A2 = NQ * MODEL_DIM
_OFF_B1 = 2 * NQ * MODEL_DIM


def _aux_tables_np():
    i = np.arange(MODEL_DIM, dtype=np.float64)
    w = 1.0 / (10000.0 ** (2.0 * i / MODEL_DIM))
    even = (np.arange(MODEL_DIM) % 2) == 0
    th = QR * np.arange(NQ, dtype=np.float64).reshape(-1, 1) * w
    a1 = np.where(even, np.sin(th), np.cos(th)).astype(np.float32)
    a2 = np.where(even, np.cos(th), -np.sin(th)).astype(np.float32)
    rw = np.arange(QR, dtype=np.float64).reshape(-1, 1) * w
    b1 = np.cos(rw).astype(np.float32)
    b2 = np.sin(rw).astype(np.float32)
    return np.concatenate(
        [a1.reshape(-1), a2.reshape(-1), b1.reshape(-1), b2.reshape(-1)]
    )


_AUX = _aux_tables_np()


def _sc_body(
    table_hbm, idx_hbm, aux_hbm, out_hbm,
    idx_v, rows_all, posrec, b_v, a_all, gsem, psem, asem, bsem, isem,
):
    wid = lax.axis_index("s") * NC + lax.axis_index("c")
    t0 = wid * T_PER_W

    bd = pltpu.async_copy(
        aux_hbm.at[pl.ds(_OFF_B1, 2 * QR * MODEL_DIM)], b_v, bsem
    )

    def issue_a(tc):
        q0 = (wid * N_TC + tc) * QPC
        buf = lax.rem(tc, 2)
        pltpu.async_copy(
            aux_hbm.at[pl.ds(_OFF_A1 + q0 * MODEL_DIM, QPC * MODEL_DIM)],
            a_all.at[buf].at[pl.ds(0, QPC * MODEL_DIM)],
            asem.at[buf],
        )
        pltpu.async_copy(
            aux_hbm.at[pl.ds(_OFF_A2 + q0 * MODEL_DIM, QPC * MODEL_DIM)],
            a_all.at[buf].at[pl.ds(QPC * MODEL_DIM, QPC * MODEL_DIM)],
            asem.at[buf],
        )

    issue_a(0)
    issue_a(1)

    idx_descs = [
        pltpu.async_copy(
            idx_hbm.at[b, pl.ds(t0, T_PER_W)],
            idx_v.at[pl.ds(b * T_PER_W, T_PER_W)],
            isem,
        )
        for b in range(BATCH)
    ]
    for d in idx_descs:
        d.wait()

    def issue_gather(n):
        tc = n // BATCH
        b = lax.rem(n, BATCH)
        jn = lax.rem(n, NB)
        pltpu.async_copy(
            table_hbm.at[idx_v.at[pl.ds(b * T_PER_W + tc * CHUNK, CHUNK)]],
            rows_all.at[jn],
            gsem.at[jn],
        )

    def wait_dma(sem, vmem_ref):
        pltpu.make_async_copy(
            table_hbm.at[pl.ds(0, CHUNK)], vmem_ref, sem
        ).wait()

    for n in range(GLEAD):
        issue_gather(n)

    def chunk_body(c, _):
        j = lax.rem(c, NB)
        tc = c // BATCH
        b = lax.rem(c, BATCH)

        n = c + GLEAD
        jn = lax.rem(n, NB)

        @pl.when(n < N_CHUNKS)
        def _():
            @pl.when(n >= NB)
            def _():
                wait_dma(psem.at[jn], rows_all.at[jn])

            issue_gather(n)

        @pl.when(b == 0)
        def _():
            buf = lax.rem(tc, 2)
            a_tc = a_all.at[buf]
            pltpu.make_async_copy(
                aux_hbm.at[pl.ds(0, 2 * QPC * MODEL_DIM)], a_tc, asem.at[buf]
            ).wait()

            @plsc.parallel_loop(0, _VR, unroll=1)
            def gen_body(jc):
                off = pl.multiple_of(jc << 4, LANES)
                a1q0 = a_tc[pl.ds(off, LANES)]
                a1q1 = a_tc[pl.ds(MODEL_DIM + off, LANES)]
                a2q0 = a_tc[pl.ds(2 * MODEL_DIM + off, LANES)]
                a2q1 = a_tc[pl.ds(3 * MODEL_DIM + off, LANES)]
                for r in range(QR):
                    b1r = b_v[pl.ds(r * MODEL_DIM + off, LANES)]
                    b2r = b_v[pl.ds((QR + r) * MODEL_DIM + off, LANES)]
                    posrec[r, pl.ds(off, LANES)] = a1q0 * b1r + a2q0 * b2r
                    posrec[QR + r, pl.ds(off, LANES)] = a1q1 * b1r + a2q1 * b2r

            @pl.when(tc < N_TC - 2)
            def _():
                issue_a(tc + 2)

        wait_dma(gsem.at[j], rows_all.at[j])
        rows_j = rows_all.at[j]

        @plsc.parallel_loop(0, CHUNK * _VR, unroll=4)
        def add_body(i):
            r = i >> 6
            off = pl.multiple_of((i & (_VR - 1)) << 4, LANES)
            rows_j[r, pl.ds(off, LANES)] = (
                rows_j[r, pl.ds(off, LANES)] + posrec[r, pl.ds(off, LANES)]
            )

        pltpu.async_copy(
            rows_j,
            out_hbm.at[pl.ds(b * MAX_LEN + t0 + tc * CHUNK, CHUNK)],
            psem.at[j],
        )
        return 0

    bd.wait()
    lax.fori_loop(0, N_CHUNKS, chunk_body, 0)

    for c in range(N_CHUNKS - NB, N_CHUNKS):
        wait_dma(psem.at[c % NB], rows_all.at[c % NB])


@jax.jit
def _embed(idx, table, aux):
    mesh = plsc.VectorSubcoreMesh(
        core_axis_name="c", subcore_axis_name="s", num_cores=NC, num_subcores=NS
    )
    scratch = [
        pltpu.VMEM((BATCH * T_PER_W,), jnp.int32),
        pltpu.VMEM((NB, CHUNK, MODEL_DIM), jnp.float32),
        pltpu.VMEM((CHUNK, MODEL_DIM), jnp.float32),
        pltpu.VMEM((2 * QR * MODEL_DIM,), jnp.float32),
        pltpu.VMEM((2, 2 * QPC * MODEL_DIM), jnp.float32),
        pltpu.SemaphoreType.DMA((NB,)),
        pltpu.SemaphoreType.DMA((NB,)),
        pltpu.SemaphoreType.DMA((2,)),
        pltpu.SemaphoreType.DMA,
        pltpu.SemaphoreType.DMA,
    ]
    fn = pl.kernel(
        _sc_body,
        out_type=jax.ShapeDtypeStruct((B_TOTAL, MODEL_DIM), jnp.float32),
        mesh=mesh,
        scratch_types=scratch,
    )
    return fn(table, idx, aux)


def kernel(x, table):
    idx = x.astype(jnp.int32)
    out = _embed(idx, table, jnp.asarray(_AUX))
    return out.reshape(BATCH, MAX_LEN, MODEL_DIM)
